# baseline (speedup 1.0000x reference)
import functools

import jax
import jax.numpy as jnp
from jax import lax
from jax.experimental import pallas as pl
from jax.experimental.pallas import tpu as pltpu
from jax.experimental.pallas import tpu_sc as plsc

NUM_HEADS = 32
Q_LEN = 2048
K_LEN = 2048
NREP = 8
VTAB_W = 4112
CHUNK = 16
THRESHOLDS = (12, 16, 23, 32, 46, 64, 91)


def _bucket16(d):
    m = jnp.abs(d)
    large = jnp.full((16,), 8, jnp.int32)
    for t in THRESHOLDS:
        large = large + jnp.where(m >= t, 1, 0)
    half = jnp.where(m < 8, m, large)
    return jnp.where(d > 0, half + 16, half)


def _sc_body(w_hbm, out_hbm, w_v, *vt_and_sem):
    vtabs = vt_and_sem[:NREP]
    sem = vt_and_sem[NREP]
    h = lax.axis_index("s") * 2 + lax.axis_index("c")
    pltpu.sync_copy(w_hbm, w_v)
    lane = lax.iota(jnp.int32, 16)
    h_vec = jnp.zeros((16,), jnp.int32) + h

    for s in range(NREP):

        def build(j, _, s=s):
            d = j * 16 + lane + (s - (Q_LEN - 1))
            vals = plsc.load_gather(w_v, [_bucket16(d), h_vec])
            vtabs[s][pl.ds(j * 16, 16)] = vals
            return 0

        lax.fori_loop(0, VTAB_W // 16, build, 0)

    ngroups = Q_LEN // 8

    def mk_copy(g, r):
        return pltpu.make_async_copy(
            vtabs[7 - r].at[pl.ds(8 * (ngroups - 1 - g), K_LEN)],
            out_hbm.at[0, h, 8 * g + r, :],
            sem,
        )

    def fire(g):
        for r in range(8):
            mk_copy(g, r).start()

    def drain(g):
        for r in range(8):
            mk_copy(g, r).wait()

    fire(0)
    fire(1)

    def row_group(g, _):
        fire(g + 2)
        drain(g)
        return 0

    lax.fori_loop(0, ngroups - 2, row_group, 0)
    drain(ngroups - 2)
    drain(ngroups - 1)


def kernel(query_len, key_len, W):
    sc_kernel = functools.partial(
        pl.kernel,
        out_type=jax.ShapeDtypeStruct(
            (1, NUM_HEADS, Q_LEN, K_LEN), jnp.float32
        ),
        mesh=plsc.VectorSubcoreMesh(core_axis_name="c", subcore_axis_name="s"),
        scratch_types=[
            pltpu.VMEM((NUM_HEADS, NUM_HEADS), jnp.float32),
        ]
        + [pltpu.VMEM((VTAB_W,), jnp.float32) for _ in range(NREP)]
        + [pltpu.SemaphoreType.DMA],
        compiler_params=pltpu.CompilerParams(
            use_tc_tiling_on_sc=False, needs_layout_passes=False
        ),
    )(_sc_body)
    return sc_kernel(W)


# speedup vs baseline: 1.9877x; 1.9877x over previous
import functools

import jax
import jax.numpy as jnp
from jax import lax
from jax.experimental import pallas as pl
from jax.experimental.pallas import tpu as pltpu
from jax.experimental.pallas import tpu_sc as plsc

NUM_HEADS = 32
NUM_BUCKETS = 32
Q_LEN = 2048
K_LEN = 2048
Q_BLOCK = 8
VTAB_W = 4224
THRESHOLDS = (12, 16, 23, 32, 46, 64, 91)


def _bucket16(d):
    m = jnp.abs(d)
    large = jnp.full((16,), 8, jnp.int32)
    for t in THRESHOLDS:
        large = large + jnp.where(m >= t, 1, 0)
    half = jnp.where(m < 8, m, large)
    return jnp.where(d > 0, half + 16, half)


def _sc_table_body(w_hbm, vtab_hbm, w_v, row_v, sem):
    h = lax.axis_index("s") * 2 + lax.axis_index("c")
    pltpu.sync_copy(w_hbm, w_v)
    lane = lax.iota(jnp.int32, 16)
    h_vec = jnp.zeros((16,), jnp.int32) + h

    def build(j, _):
        d = j * 16 + lane - (Q_LEN - 1)
        row_v[pl.ds(j * 16, 16)] = plsc.load_gather(
            w_v, [_bucket16(d), h_vec]
        )
        return 0

    lax.fori_loop(0, VTAB_W // 16, build, 0)
    pltpu.sync_copy(row_v, vtab_hbm.at[h, :])


def _sc_table(W):
    return functools.partial(
        pl.kernel,
        out_type=jax.ShapeDtypeStruct((NUM_HEADS, VTAB_W), jnp.float32),
        mesh=plsc.VectorSubcoreMesh(core_axis_name="c", subcore_axis_name="s"),
        scratch_types=[
            pltpu.VMEM((NUM_HEADS, NUM_BUCKETS), jnp.float32),
            pltpu.VMEM((VTAB_W,), jnp.float32),
            pltpu.SemaphoreType.DMA,
        ],
        compiler_params=pltpu.CompilerParams(
            use_tc_tiling_on_sc=False, needs_layout_passes=False
        ),
    )(_sc_table_body)(W)


def _tc_expand_body(vtab_ref, o_ref):
    i = pl.program_id(0)
    for r in range(Q_BLOCK):
        start = (Q_LEN - 1) - (i * Q_BLOCK + r)
        base = (start // 128) * 128
        sub = start - base
        win = vtab_ref[:, pl.ds(base, K_LEN + 128)]
        o_ref[0, :, r, :] = pltpu.roll(win, -sub, 1)[:, :K_LEN]


def kernel(query_len, key_len, W):
    vtab = _sc_table(W)
    return pl.pallas_call(
        _tc_expand_body,
        grid=(Q_LEN // Q_BLOCK,),
        in_specs=[pl.BlockSpec((NUM_HEADS, VTAB_W), lambda i: (0, 0))],
        out_specs=pl.BlockSpec(
            (1, NUM_HEADS, Q_BLOCK, K_LEN), lambda i: (0, 0, i, 0)
        ),
        out_shape=jax.ShapeDtypeStruct(
            (1, NUM_HEADS, Q_LEN, K_LEN), jnp.float32
        ),
    )(vtab)
